# Initial kernel scaffold; baseline (speedup 1.0000x reference)
import jax
import jax.numpy as jnp
from jax.experimental import pallas as pl


def kernel(x_prev, rand_node_features, t_idx_per_node, edge_index, W_enc1, b_enc1, W_enc2, b_enc2, W_msg, b_msg, W_upd1, b_upd1, W_upd2, b_upd2, W_dec1, b_dec1, W_dec2, b_dec2, W_head1, b_head1, W_head2, b_head2):
    raise NotImplementedError("write your pallas kernel here")


# speedup vs baseline: 6.6051x; 6.6051x over previous
import functools

import jax
import jax.numpy as jnp
import numpy as np
from jax import lax
from jax.experimental import pallas as pl
from jax.experimental.pallas import tpu as pltpu
from jax.experimental.pallas import tpu_sc as plsc

N = 100000
E = 1600000
H = 48
EMB = 32
TMAX = 100
NMP = 8
NBERN = 2
NRAND = 5

NC = 2
NS = 16
NW = NC * NS
L = 16
CG = H // L

EPT = E // NW
K = 1000
NCHUNK = EPT // K
NPAD = 102400
NPS = NPAD // NS
ZR = 320
NZ = NPS // ZR
NB = 1000
GRID = N // NB

@functools.cache
def _sc_kernels():
    mesh = plsc.VectorSubcoreMesh(
        core_axis_name="c", subcore_axis_name="s", num_cores=NC, num_subcores=NS)
    params = pltpu.CompilerParams(use_tc_tiling_on_sc=False)

    @functools.partial(
        pl.kernel,
        out_type=jax.ShapeDtypeStruct((NC, CG, NPAD, L), jnp.float32),
        mesh=mesh,
        compiler_params=params,
        scratch_types=[
            pltpu.VMEM((K,), jnp.int32),
            pltpu.VMEM((K,), jnp.int32),
            pltpu.VMEM((K, L), jnp.float32),
            pltpu.VMEM_SHARED((NPAD, L), jnp.float32),
            pltpu.VMEM((ZR, L), jnp.float32),
            pltpu.SemaphoreType.DMA,
        ],
    )
    def sc_round(h0, h1, h2, snd, rcv, zeros_h, out, sidx, ridx, rows, acc, zbuf, sem):
        sid = lax.axis_index("s")
        core = lax.axis_index("c")
        wid = core * NS + sid
        pltpu.sync_copy(zeros_h, zbuf)
        for cg, hg in enumerate((h0, h1, h2)):
            def zbody(z, carry):
                pltpu.sync_copy(zbuf, acc.at[pl.ds(sid * NPS + z * ZR, ZR)])
                return carry

            lax.fori_loop(0, NZ, zbody, 0)
            plsc.subcore_barrier()

            def cbody(c, carry):
                off = (wid * NCHUNK + c) * K
                pltpu.sync_copy(snd.at[pl.ds(off, K)], sidx)
                pltpu.sync_copy(rcv.at[pl.ds(off, K)], ridx)
                pltpu.async_copy(hg.at[sidx], rows, sem).wait()
                pltpu.sync_copy(rows, acc.at[ridx], add=True)
                return carry

            lax.fori_loop(0, NCHUNK, cbody, 0)
            plsc.subcore_barrier()

            pltpu.sync_copy(
                acc.at[pl.ds(sid * NPS, NPS)],
                out.at[core, cg, pl.ds(sid * NPS, NPS)],
            )

    @functools.partial(
        pl.kernel,
        out_type=jax.ShapeDtypeStruct((NC, NPAD, 1), jnp.float32),
        mesh=mesh,
        compiler_params=params,
        scratch_types=[
            pltpu.VMEM((K,), jnp.int32),
            pltpu.VMEM((K, 1), jnp.float32),
            pltpu.VMEM_SHARED((NPAD, 1), jnp.float32),
            pltpu.VMEM((NPS, 1), jnp.float32),
        ],
    )
    def sc_deg(rcv, ones_h, zeros_h, out, ridx, ones_v, acc, zbuf):
        sid = lax.axis_index("s")
        core = lax.axis_index("c")
        wid = core * NS + sid
        pltpu.sync_copy(ones_h, ones_v)
        pltpu.sync_copy(zeros_h, zbuf)
        pltpu.sync_copy(zbuf, acc.at[pl.ds(sid * NPS, NPS)])
        plsc.subcore_barrier()

        def cbody(c, carry):
            off = (wid * NCHUNK + c) * K
            pltpu.sync_copy(rcv.at[pl.ds(off, K)], ridx)
            pltpu.sync_copy(ones_v, acc.at[ridx], add=True)
            return carry

        lax.fori_loop(0, NCHUNK, cbody, 0)
        plsc.subcore_barrier()
        pltpu.sync_copy(acc.at[pl.ds(sid * NPS, NPS)],
                        out.at[core, pl.ds(sid * NPS, NPS)])


    return sc_round, sc_deg


def _sc_round(*args):
    return _sc_kernels()[0](*args)


def _sc_deg(*args):
    return _sc_kernels()[1](*args)


def _relu(x):
    return jnp.maximum(x, 0.0)


def _dot(a, b):
    return jnp.dot(a, b, preferred_element_type=jnp.float32)


def _enc_body(x_ref, t_ref, r_ref, div_ref, w1_ref, b1_ref, w2_ref, b2_ref,
              o0, o1, o2):
    x = x_ref[...]
    t = t_ref[...].astype(jnp.float32)
    arg = t * div_ref[...]
    f = jnp.concatenate(
        [
            (x == 0).astype(jnp.float32),
            (x == 1).astype(jnp.float32),
            jnp.sin(arg),
            jnp.cos(arg),
            r_ref[...],
        ],
        axis=-1,
    )
    h = _relu(_dot(f, w1_ref[...]) + b1_ref[...])
    h = _relu(_dot(h, w2_ref[...]) + b2_ref[...])
    o0[...] = h[:, 0 * L:1 * L]
    o1[...] = h[:, 1 * L:2 * L]
    o2[...] = h[:, 2 * L:3 * L]


def _round_body(h0_ref, h1_ref, h2_ref, p_ref, d0_ref, d1_ref,
                w1h_ref, wm1_ref, bm1_ref, bu1_ref, wu2_ref, bu2_ref,
                o0, o1, o2):
    hb = jnp.concatenate([h0_ref[...], h1_ref[...], h2_ref[...]], axis=-1)
    p = p_ref[...]
    agg0 = jnp.concatenate(
        [p[0, 0] + p[1, 0], p[0, 1] + p[1, 1], p[0, 2] + p[1, 2]], axis=-1)
    deg = d0_ref[...] + d1_ref[...]
    t1 = _relu(_dot(hb, w1h_ref[...]) + _dot(agg0, wm1_ref[...])
               + deg * bm1_ref[...] + bu1_ref[...])
    hn = _relu(_dot(t1, wu2_ref[...]) + bu2_ref[...])
    o0[...] = hn[:, 0 * L:1 * L]
    o1[...] = hn[:, 1 * L:2 * L]
    o2[...] = hn[:, 2 * L:3 * L]


def _dec_body(h0_ref, h1_ref, h2_ref, wd1_ref, bd1_ref, wd2_ref, bd2_ref,
              wh1_ref, bh1_ref, wh2_ref, bh2_ref, o_ref):
    hb = jnp.concatenate([h0_ref[...], h1_ref[...], h2_ref[...]], axis=-1)
    d = _relu(_dot(hb, wd1_ref[...]) + bd1_ref[...])
    d = _dot(d, wd2_ref[...]) + bd2_ref[...]
    s = _relu(_dot(d, wh1_ref[...]) + bh1_ref[...])
    o_ref[...] = _dot(s, wh2_ref[...]) + bh2_ref[...]


def _row_spec(width):
    return pl.BlockSpec((NB, width), lambda i: (i, 0))


def _full_spec(shape):
    nd = len(shape)
    return pl.BlockSpec(shape, lambda i, _n=nd: (0,) * _n)


_H_SPECS = [_row_spec(L), _row_spec(L), _row_spec(L)]


def kernel(x_prev, rand_node_features, t_idx_per_node, edge_index,
           W_enc1, b_enc1, W_enc2, b_enc2, W_msg, b_msg,
           W_upd1, b_upd1, W_upd2, b_upd2, W_dec1, b_dec1, W_dec2, b_dec2,
           W_head1, b_head1, W_head2, b_head2):
    f32 = jnp.float32

    senders = edge_index[0].astype(jnp.int32)
    receivers = edge_index[1].astype(jnp.int32)

    div = jnp.exp(
        jnp.arange(0, EMB, 2, dtype=f32) * (-np.log(float(TMAX)) / EMB)
    ).reshape(1, EMB // 2)

    w1h = W_upd1[:H]
    w1a = W_upd1[H:]
    wm1 = _dot(W_msg, w1a)
    bm1 = _dot(b_msg.reshape(1, H), w1a)

    zeros2 = jnp.zeros((ZR, L), f32)
    zeros1 = jnp.zeros((NPS, 1), f32)
    ones2 = jnp.ones((K, 1), f32)

    row1 = lambda b: b.reshape(1, -1)

    h0, h1, h2 = pl.pallas_call(
        _enc_body,
        grid=(GRID,),
        in_specs=[
            _row_spec(1), _row_spec(1), _row_spec(NRAND),
            _full_spec((1, EMB // 2)),
            _full_spec(W_enc1.shape), _full_spec((1, H)),
            _full_spec(W_enc2.shape), _full_spec((1, H)),
        ],
        out_specs=_H_SPECS,
        out_shape=[jax.ShapeDtypeStruct((N, L), f32)] * 3,
    )(x_prev, t_idx_per_node, rand_node_features, div,
      W_enc1, row1(b_enc1), W_enc2, row1(b_enc2))

    degp = _sc_deg(receivers, ones2, zeros1)
    deg0 = degp[0, :N]
    deg1 = degp[1, :N]

    dense_round = pl.pallas_call(
        _round_body,
        grid=(GRID,),
        in_specs=[
            _row_spec(L), _row_spec(L), _row_spec(L),
            pl.BlockSpec((NC, CG, NB, L), lambda i: (0, 0, i, 0)),
            _row_spec(1), _row_spec(1),
            _full_spec((H, H)), _full_spec((H, H)), _full_spec((1, H)),
            _full_spec((1, H)), _full_spec((H, H)), _full_spec((1, H)),
        ],
        out_specs=_H_SPECS,
        out_shape=[jax.ShapeDtypeStruct((N, L), f32)] * 3,
    )

    for _ in range(NMP):
        p = _sc_round(h0, h1, h2, senders, receivers, zeros2)
        h0, h1, h2 = dense_round(h0, h1, h2, p, deg0, deg1,
                                 w1h, wm1, bm1, row1(b_upd1),
                                 W_upd2, row1(b_upd2))

    out = pl.pallas_call(
        _dec_body,
        grid=(GRID,),
        in_specs=[
            _row_spec(L), _row_spec(L), _row_spec(L),
            _full_spec((H, H)), _full_spec((1, H)),
            _full_spec((H, H)), _full_spec((1, H)),
            _full_spec((H, H)), _full_spec((1, H)),
            _full_spec((H, NBERN)), _full_spec((1, NBERN)),
        ],
        out_specs=_row_spec(NBERN),
        out_shape=jax.ShapeDtypeStruct((N, NBERN), f32),
    )(h0, h1, h2, W_dec1, row1(b_dec1), W_dec2, row1(b_dec2),
      W_head1, row1(b_head1), W_head2, row1(b_head2))

    return out.reshape(N, 1, NBERN)


# speedup vs baseline: 6.7358x; 1.0198x over previous
import functools

import jax
import jax.numpy as jnp
import numpy as np
from jax import lax
from jax.experimental import pallas as pl
from jax.experimental.pallas import tpu as pltpu
from jax.experimental.pallas import tpu_sc as plsc

N = 100000
E = 1600000
H = 48
EMB = 32
TMAX = 100
NMP = 8
NBERN = 2
NRAND = 5

NC = 2
NS = 16
NW = NC * NS
L = 16
CG = H // L

EPT = E // NW
K = 400
NCHUNK = EPT // K
NPAD = 102400
NPS = NPAD // NS
ZR = 320
NZ = NPS // ZR
NB = 1000
GRID = N // NB

@functools.cache
def _sc_kernels():
    mesh = plsc.VectorSubcoreMesh(
        core_axis_name="c", subcore_axis_name="s", num_cores=NC, num_subcores=NS)
    params = pltpu.CompilerParams(use_tc_tiling_on_sc=False)

    @functools.partial(
        pl.kernel,
        out_type=jax.ShapeDtypeStruct((NC, CG, NPAD, L), jnp.float32),
        mesh=mesh,
        compiler_params=params,
        scratch_types=[
            pltpu.VMEM((2, K), jnp.int32),
            pltpu.VMEM((2, K), jnp.int32),
            pltpu.VMEM((2, K, L), jnp.float32),
            pltpu.VMEM_SHARED((NPAD, L), jnp.float32),
            pltpu.VMEM((ZR, L), jnp.float32),
            pltpu.SemaphoreType.DMA,
            pltpu.SemaphoreType.DMA,
        ],
    )
    def sc_round(h0, h1, h2, snd, rcv, zeros_h, out, sidx, ridx, rows, acc, zbuf,
                 sem0, sem1):
        sid = lax.axis_index("s")
        core = lax.axis_index("c")
        wid = core * NS + sid
        pltpu.sync_copy(zeros_h, zbuf)
        for cg, hg in enumerate((h0, h1, h2)):
            def zbody(z, carry):
                pltpu.sync_copy(zbuf, acc.at[pl.ds(sid * NPS + z * ZR, ZR)])
                return carry

            lax.fori_loop(0, NZ, zbody, 0)
            plsc.subcore_barrier()

            def start(c, buf, sem_):
                off = (wid * NCHUNK + jnp.minimum(c, NCHUNK - 1)) * K
                pltpu.sync_copy(snd.at[pl.ds(off, K)], sidx.at[buf])
                pltpu.sync_copy(rcv.at[pl.ds(off, K)], ridx.at[buf])
                pltpu.async_copy(hg.at[sidx.at[buf]], rows.at[buf], sem_)

            def wait(buf, sem_):
                pltpu.make_async_copy(
                    hg.at[sidx.at[buf]], rows.at[buf], sem_).wait()

            def scat(buf):
                pltpu.sync_copy(rows.at[buf], acc.at[ridx.at[buf]], add=True)

            start(0, 0, sem0)

            def cbody(c, carry):
                c2 = 2 * c
                start(c2 + 1, 1, sem1)
                wait(0, sem0)
                scat(0)
                start(c2 + 2, 0, sem0)
                wait(1, sem1)
                scat(1)
                return carry

            lax.fori_loop(0, NCHUNK // 2, cbody, 0)
            wait(0, sem0)
            scat(0)
            plsc.subcore_barrier()

            pltpu.sync_copy(
                acc.at[pl.ds(sid * NPS, NPS)],
                out.at[core, cg, pl.ds(sid * NPS, NPS)],
            )

    @functools.partial(
        pl.kernel,
        out_type=jax.ShapeDtypeStruct((NC, NPAD, 1), jnp.float32),
        mesh=mesh,
        compiler_params=params,
        scratch_types=[
            pltpu.VMEM((K,), jnp.int32),
            pltpu.VMEM((K, 1), jnp.float32),
            pltpu.VMEM_SHARED((NPAD, 1), jnp.float32),
            pltpu.VMEM((NPS, 1), jnp.float32),
        ],
    )
    def sc_deg(rcv, ones_h, zeros_h, out, ridx, ones_v, acc, zbuf):
        sid = lax.axis_index("s")
        core = lax.axis_index("c")
        wid = core * NS + sid
        pltpu.sync_copy(ones_h, ones_v)
        pltpu.sync_copy(zeros_h, zbuf)
        pltpu.sync_copy(zbuf, acc.at[pl.ds(sid * NPS, NPS)])
        plsc.subcore_barrier()

        def cbody(c, carry):
            off = (wid * NCHUNK + c) * K
            pltpu.sync_copy(rcv.at[pl.ds(off, K)], ridx)
            pltpu.sync_copy(ones_v, acc.at[ridx], add=True)
            return carry

        lax.fori_loop(0, NCHUNK, cbody, 0)
        plsc.subcore_barrier()
        pltpu.sync_copy(acc.at[pl.ds(sid * NPS, NPS)],
                        out.at[core, pl.ds(sid * NPS, NPS)])


    return sc_round, sc_deg


def _sc_round(*args):
    return _sc_kernels()[0](*args)


def _sc_deg(*args):
    return _sc_kernels()[1](*args)


def _relu(x):
    return jnp.maximum(x, 0.0)


def _dot(a, b):
    return jnp.dot(a, b, preferred_element_type=jnp.float32)


def _enc_body(x_ref, t_ref, r_ref, div_ref, w1_ref, b1_ref, w2_ref, b2_ref,
              o0, o1, o2):
    x = x_ref[...]
    t = t_ref[...].astype(jnp.float32)
    arg = t * div_ref[...]
    f = jnp.concatenate(
        [
            (x == 0).astype(jnp.float32),
            (x == 1).astype(jnp.float32),
            jnp.sin(arg),
            jnp.cos(arg),
            r_ref[...],
        ],
        axis=-1,
    )
    h = _relu(_dot(f, w1_ref[...]) + b1_ref[...])
    h = _relu(_dot(h, w2_ref[...]) + b2_ref[...])
    o0[...] = h[:, 0 * L:1 * L]
    o1[...] = h[:, 1 * L:2 * L]
    o2[...] = h[:, 2 * L:3 * L]


def _round_body(h0_ref, h1_ref, h2_ref, p_ref, d0_ref, d1_ref,
                w1h_ref, wm1_ref, bm1_ref, bu1_ref, wu2_ref, bu2_ref,
                o0, o1, o2):
    hb = jnp.concatenate([h0_ref[...], h1_ref[...], h2_ref[...]], axis=-1)
    p = p_ref[...]
    agg0 = jnp.concatenate(
        [p[0, 0] + p[1, 0], p[0, 1] + p[1, 1], p[0, 2] + p[1, 2]], axis=-1)
    deg = d0_ref[...] + d1_ref[...]
    t1 = _relu(_dot(hb, w1h_ref[...]) + _dot(agg0, wm1_ref[...])
               + deg * bm1_ref[...] + bu1_ref[...])
    hn = _relu(_dot(t1, wu2_ref[...]) + bu2_ref[...])
    o0[...] = hn[:, 0 * L:1 * L]
    o1[...] = hn[:, 1 * L:2 * L]
    o2[...] = hn[:, 2 * L:3 * L]


def _dec_body(h0_ref, h1_ref, h2_ref, wd1_ref, bd1_ref, wd2_ref, bd2_ref,
              wh1_ref, bh1_ref, wh2_ref, bh2_ref, o_ref):
    hb = jnp.concatenate([h0_ref[...], h1_ref[...], h2_ref[...]], axis=-1)
    d = _relu(_dot(hb, wd1_ref[...]) + bd1_ref[...])
    d = _dot(d, wd2_ref[...]) + bd2_ref[...]
    s = _relu(_dot(d, wh1_ref[...]) + bh1_ref[...])
    o_ref[...] = _dot(s, wh2_ref[...]) + bh2_ref[...]


def _row_spec(width):
    return pl.BlockSpec((NB, width), lambda i: (i, 0))


def _full_spec(shape):
    nd = len(shape)
    return pl.BlockSpec(shape, lambda i, _n=nd: (0,) * _n)


_H_SPECS = [_row_spec(L), _row_spec(L), _row_spec(L)]


def kernel(x_prev, rand_node_features, t_idx_per_node, edge_index,
           W_enc1, b_enc1, W_enc2, b_enc2, W_msg, b_msg,
           W_upd1, b_upd1, W_upd2, b_upd2, W_dec1, b_dec1, W_dec2, b_dec2,
           W_head1, b_head1, W_head2, b_head2):
    f32 = jnp.float32

    senders = edge_index[0].astype(jnp.int32)
    receivers = edge_index[1].astype(jnp.int32)

    div = jnp.exp(
        jnp.arange(0, EMB, 2, dtype=f32) * (-np.log(float(TMAX)) / EMB)
    ).reshape(1, EMB // 2)

    w1h = W_upd1[:H]
    w1a = W_upd1[H:]
    wm1 = _dot(W_msg, w1a)
    bm1 = _dot(b_msg.reshape(1, H), w1a)

    zeros2 = jnp.zeros((ZR, L), f32)
    zeros1 = jnp.zeros((NPS, 1), f32)
    ones2 = jnp.ones((K, 1), f32)

    row1 = lambda b: b.reshape(1, -1)

    h0, h1, h2 = pl.pallas_call(
        _enc_body,
        grid=(GRID,),
        in_specs=[
            _row_spec(1), _row_spec(1), _row_spec(NRAND),
            _full_spec((1, EMB // 2)),
            _full_spec(W_enc1.shape), _full_spec((1, H)),
            _full_spec(W_enc2.shape), _full_spec((1, H)),
        ],
        out_specs=_H_SPECS,
        out_shape=[jax.ShapeDtypeStruct((N, L), f32)] * 3,
    )(x_prev, t_idx_per_node, rand_node_features, div,
      W_enc1, row1(b_enc1), W_enc2, row1(b_enc2))

    degp = _sc_deg(receivers, ones2, zeros1)
    deg0 = degp[0, :N]
    deg1 = degp[1, :N]

    dense_round = pl.pallas_call(
        _round_body,
        grid=(GRID,),
        in_specs=[
            _row_spec(L), _row_spec(L), _row_spec(L),
            pl.BlockSpec((NC, CG, NB, L), lambda i: (0, 0, i, 0)),
            _row_spec(1), _row_spec(1),
            _full_spec((H, H)), _full_spec((H, H)), _full_spec((1, H)),
            _full_spec((1, H)), _full_spec((H, H)), _full_spec((1, H)),
        ],
        out_specs=_H_SPECS,
        out_shape=[jax.ShapeDtypeStruct((N, L), f32)] * 3,
    )

    for _ in range(NMP):
        p = _sc_round(h0, h1, h2, senders, receivers, zeros2)
        h0, h1, h2 = dense_round(h0, h1, h2, p, deg0, deg1,
                                 w1h, wm1, bm1, row1(b_upd1),
                                 W_upd2, row1(b_upd2))

    out = pl.pallas_call(
        _dec_body,
        grid=(GRID,),
        in_specs=[
            _row_spec(L), _row_spec(L), _row_spec(L),
            _full_spec((H, H)), _full_spec((1, H)),
            _full_spec((H, H)), _full_spec((1, H)),
            _full_spec((H, H)), _full_spec((1, H)),
            _full_spec((H, NBERN)), _full_spec((1, NBERN)),
        ],
        out_specs=_row_spec(NBERN),
        out_shape=jax.ShapeDtypeStruct((N, NBERN), f32),
    )(h0, h1, h2, W_dec1, row1(b_dec1), W_dec2, row1(b_dec2),
      W_head1, row1(b_head1), W_head2, row1(b_head2))

    return out.reshape(N, 1, NBERN)


# speedup vs baseline: 8.4609x; 1.2561x over previous
import functools

import jax
import jax.numpy as jnp
import numpy as np
from jax import lax
from jax.experimental import pallas as pl
from jax.experimental.pallas import tpu as pltpu
from jax.experimental.pallas import tpu_sc as plsc

N = 100000
E = 1600000
H = 48
EMB = 32
TMAX = 100
NMP = 8
NBERN = 2
NRAND = 5

NC = 2
NS = 16
NW = NC * NS
L = 16
CG = H // L

EPT = E // NW
K = 400
NCHUNK = EPT // K
NPAD = 102400
NPS = NPAD // NS
ZR = 320
NZ = NPS // ZR
NB = 1000
GRID = N // NB

@functools.cache
def _sc_kernels():
    mesh = plsc.VectorSubcoreMesh(
        core_axis_name="c", subcore_axis_name="s", num_cores=NC, num_subcores=NS)
    params = pltpu.CompilerParams(use_tc_tiling_on_sc=False)

    @functools.partial(
        pl.kernel,
        out_type=jax.ShapeDtypeStruct((NC, CG, NPAD, L), jnp.float32),
        mesh=mesh,
        compiler_params=params,
        scratch_types=[
            pltpu.VMEM((3, K), jnp.int32),
            pltpu.VMEM((3, K), jnp.int32),
            pltpu.VMEM((2, K, L), jnp.float32),
            pltpu.VMEM_SHARED((NPAD, L), jnp.float32),
            pltpu.VMEM((ZR, L), jnp.float32),
            pltpu.SemaphoreType.DMA((3,)),
            pltpu.SemaphoreType.DMA((2,)),
            pltpu.SemaphoreType.DMA((2,)),
        ],
    )
    def sc_round(h0, h1, h2, snd, rcv, zeros_h, out, sidx, ridx, rows, acc, zbuf,
                 semi, semg, sems):
        sid = lax.axis_index("s")
        core = lax.axis_index("c")
        wid = core * NS + sid
        pltpu.sync_copy(zeros_h, zbuf)
        for cg, hg in enumerate((h0, h1, h2)):
            def zbody(z, carry):
                pltpu.sync_copy(zbuf, acc.at[pl.ds(sid * NPS + z * ZR, ZR)])
                return carry

            lax.fori_loop(0, NZ, zbody, 0)
            plsc.subcore_barrier()

            def idx_start(c, ib):
                cc = jnp.minimum(c, NCHUNK - 1)
                row = wid * NCHUNK + cc
                pltpu.async_copy(snd.at[row], sidx.at[ib], semi.at[ib])
                pltpu.async_copy(rcv.at[row], ridx.at[ib], semi.at[ib])

            def idx_wait(c, ib):
                cc = jnp.minimum(c, NCHUNK - 1)
                row = wid * NCHUNK + cc
                pltpu.make_async_copy(snd.at[row], sidx.at[ib], semi.at[ib]).wait()
                pltpu.make_async_copy(rcv.at[row], ridx.at[ib], semi.at[ib]).wait()

            def gath_start(ib, b):
                pltpu.async_copy(hg.at[sidx.at[ib]], rows.at[b], semg.at[b])

            def gath_wait(ib, b):
                pltpu.make_async_copy(
                    hg.at[sidx.at[ib]], rows.at[b], semg.at[b]).wait()

            def scat_start(ib, b):
                pltpu.async_copy(rows.at[b], acc.at[ridx.at[ib]], sems.at[b],
                                 add=True)

            def scat_wait(ib, b):
                pltpu.make_async_copy(rows.at[b], acc.at[ridx.at[ib]],
                                      sems.at[b]).wait()

            idx_start(0, 0)

            def cbody(c, carry):
                ib = lax.rem(c, 3)
                ibn = lax.rem(c + 1, 3)
                b = lax.rem(c, 2)
                nb = lax.rem(c + 1, 2)
                idx_wait(c, ib)

                @pl.when(c >= 2)
                def _():
                    scat_wait(lax.rem(c - 2, 3), b)

                gath_start(ib, b)
                idx_start(c + 1, ibn)

                @pl.when(c >= 1)
                def _():
                    gath_wait(lax.rem(c - 1, 3), nb)
                    scat_start(lax.rem(c - 1, 3), nb)

                return carry

            lax.fori_loop(0, NCHUNK, cbody, 0)
            lastb = (NCHUNK - 1) % 2
            lasti = (NCHUNK - 1) % 3
            gath_wait(lasti, lastb)
            scat_start(lasti, lastb)
            scat_wait((NCHUNK - 2) % 3, (NCHUNK - 2) % 2)
            scat_wait(lasti, lastb)
            idx_wait(NCHUNK, NCHUNK % 3)
            plsc.subcore_barrier()

            pltpu.sync_copy(
                acc.at[pl.ds(sid * NPS, NPS)],
                out.at[core, cg, pl.ds(sid * NPS, NPS)],
            )

    @functools.partial(
        pl.kernel,
        out_type=jax.ShapeDtypeStruct((NC, NPAD, 1), jnp.float32),
        mesh=mesh,
        compiler_params=params,
        scratch_types=[
            pltpu.VMEM((K,), jnp.int32),
            pltpu.VMEM((K, 1), jnp.float32),
            pltpu.VMEM_SHARED((NPAD, 1), jnp.float32),
            pltpu.VMEM((NPS, 1), jnp.float32),
        ],
    )
    def sc_deg(rcv, ones_h, zeros_h, out, ridx, ones_v, acc, zbuf):
        sid = lax.axis_index("s")
        core = lax.axis_index("c")
        wid = core * NS + sid
        pltpu.sync_copy(ones_h, ones_v)
        pltpu.sync_copy(zeros_h, zbuf)
        pltpu.sync_copy(zbuf, acc.at[pl.ds(sid * NPS, NPS)])
        plsc.subcore_barrier()

        def cbody(c, carry):
            pltpu.sync_copy(rcv.at[wid * NCHUNK + c], ridx)
            pltpu.sync_copy(ones_v, acc.at[ridx], add=True)
            return carry

        lax.fori_loop(0, NCHUNK, cbody, 0)
        plsc.subcore_barrier()
        pltpu.sync_copy(acc.at[pl.ds(sid * NPS, NPS)],
                        out.at[core, pl.ds(sid * NPS, NPS)])


    return sc_round, sc_deg


def _sc_round(*args):
    return _sc_kernels()[0](*args)


def _sc_deg(*args):
    return _sc_kernels()[1](*args)


def _relu(x):
    return jnp.maximum(x, 0.0)


def _dot(a, b):
    return jnp.dot(a, b, preferred_element_type=jnp.float32)


def _enc_body(x_ref, t_ref, r_ref, div_ref, w1_ref, b1_ref, w2_ref, b2_ref,
              o0, o1, o2):
    x = x_ref[...]
    t = t_ref[...].astype(jnp.float32)
    arg = t * div_ref[...]
    f = jnp.concatenate(
        [
            (x == 0).astype(jnp.float32),
            (x == 1).astype(jnp.float32),
            jnp.sin(arg),
            jnp.cos(arg),
            r_ref[...],
        ],
        axis=-1,
    )
    h = _relu(_dot(f, w1_ref[...]) + b1_ref[...])
    h = _relu(_dot(h, w2_ref[...]) + b2_ref[...])
    o0[...] = h[:, 0 * L:1 * L]
    o1[...] = h[:, 1 * L:2 * L]
    o2[...] = h[:, 2 * L:3 * L]


def _round_body(h0_ref, h1_ref, h2_ref, p_ref, d0_ref, d1_ref,
                w1h_ref, wm1_ref, bm1_ref, bu1_ref, wu2_ref, bu2_ref,
                o0, o1, o2):
    hb = jnp.concatenate([h0_ref[...], h1_ref[...], h2_ref[...]], axis=-1)
    p = p_ref[...]
    agg0 = jnp.concatenate(
        [p[0, 0] + p[1, 0], p[0, 1] + p[1, 1], p[0, 2] + p[1, 2]], axis=-1)
    deg = d0_ref[...] + d1_ref[...]
    t1 = _relu(_dot(hb, w1h_ref[...]) + _dot(agg0, wm1_ref[...])
               + deg * bm1_ref[...] + bu1_ref[...])
    hn = _relu(_dot(t1, wu2_ref[...]) + bu2_ref[...])
    o0[...] = hn[:, 0 * L:1 * L]
    o1[...] = hn[:, 1 * L:2 * L]
    o2[...] = hn[:, 2 * L:3 * L]


def _dec_body(h0_ref, h1_ref, h2_ref, wd1_ref, bd1_ref, wd2_ref, bd2_ref,
              wh1_ref, bh1_ref, wh2_ref, bh2_ref, o_ref):
    hb = jnp.concatenate([h0_ref[...], h1_ref[...], h2_ref[...]], axis=-1)
    d = _relu(_dot(hb, wd1_ref[...]) + bd1_ref[...])
    d = _dot(d, wd2_ref[...]) + bd2_ref[...]
    s = _relu(_dot(d, wh1_ref[...]) + bh1_ref[...])
    o_ref[...] = _dot(s, wh2_ref[...]) + bh2_ref[...]


def _row_spec(width):
    return pl.BlockSpec((NB, width), lambda i: (i, 0))


def _full_spec(shape):
    nd = len(shape)
    return pl.BlockSpec(shape, lambda i, _n=nd: (0,) * _n)


_H_SPECS = [_row_spec(L), _row_spec(L), _row_spec(L)]


def kernel(x_prev, rand_node_features, t_idx_per_node, edge_index,
           W_enc1, b_enc1, W_enc2, b_enc2, W_msg, b_msg,
           W_upd1, b_upd1, W_upd2, b_upd2, W_dec1, b_dec1, W_dec2, b_dec2,
           W_head1, b_head1, W_head2, b_head2):
    f32 = jnp.float32

    senders = edge_index[0].astype(jnp.int32).reshape(E // K, K)
    receivers = edge_index[1].astype(jnp.int32).reshape(E // K, K)

    div = jnp.exp(
        jnp.arange(0, EMB, 2, dtype=f32) * (-np.log(float(TMAX)) / EMB)
    ).reshape(1, EMB // 2)

    w1h = W_upd1[:H]
    w1a = W_upd1[H:]
    wm1 = _dot(W_msg, w1a)
    bm1 = _dot(b_msg.reshape(1, H), w1a)

    zeros2 = jnp.zeros((ZR, L), f32)
    zeros1 = jnp.zeros((NPS, 1), f32)
    ones2 = jnp.ones((K, 1), f32)

    row1 = lambda b: b.reshape(1, -1)

    h0, h1, h2 = pl.pallas_call(
        _enc_body,
        grid=(GRID,),
        in_specs=[
            _row_spec(1), _row_spec(1), _row_spec(NRAND),
            _full_spec((1, EMB // 2)),
            _full_spec(W_enc1.shape), _full_spec((1, H)),
            _full_spec(W_enc2.shape), _full_spec((1, H)),
        ],
        out_specs=_H_SPECS,
        out_shape=[jax.ShapeDtypeStruct((N, L), f32)] * 3,
    )(x_prev, t_idx_per_node, rand_node_features, div,
      W_enc1, row1(b_enc1), W_enc2, row1(b_enc2))

    degp = _sc_deg(receivers, ones2, zeros1)
    deg0 = degp[0, :N]
    deg1 = degp[1, :N]

    dense_round = pl.pallas_call(
        _round_body,
        grid=(GRID,),
        in_specs=[
            _row_spec(L), _row_spec(L), _row_spec(L),
            pl.BlockSpec((NC, CG, NB, L), lambda i: (0, 0, i, 0)),
            _row_spec(1), _row_spec(1),
            _full_spec((H, H)), _full_spec((H, H)), _full_spec((1, H)),
            _full_spec((1, H)), _full_spec((H, H)), _full_spec((1, H)),
        ],
        out_specs=_H_SPECS,
        out_shape=[jax.ShapeDtypeStruct((N, L), f32)] * 3,
    )

    for _ in range(NMP):
        p = _sc_round(h0, h1, h2, senders, receivers, zeros2)
        h0, h1, h2 = dense_round(h0, h1, h2, p, deg0, deg1,
                                 w1h, wm1, bm1, row1(b_upd1),
                                 W_upd2, row1(b_upd2))

    out = pl.pallas_call(
        _dec_body,
        grid=(GRID,),
        in_specs=[
            _row_spec(L), _row_spec(L), _row_spec(L),
            _full_spec((H, H)), _full_spec((1, H)),
            _full_spec((H, H)), _full_spec((1, H)),
            _full_spec((H, H)), _full_spec((1, H)),
            _full_spec((H, NBERN)), _full_spec((1, NBERN)),
        ],
        out_specs=_row_spec(NBERN),
        out_shape=jax.ShapeDtypeStruct((N, NBERN), f32),
    )(h0, h1, h2, W_dec1, row1(b_dec1), W_dec2, row1(b_dec2),
      W_head1, row1(b_head1), W_head2, row1(b_head2))

    return out.reshape(N, 1, NBERN)
